# Initial kernel scaffold; baseline (speedup 1.0000x reference)
import jax
import jax.numpy as jnp
from jax.experimental import pallas as pl


def kernel(u, i, neighbors, entity_table, user_table, W, a):
    raise NotImplementedError("write your pallas kernel here")


# speedup vs baseline: 1.3557x; 1.3557x over previous
import functools

import jax
import jax.numpy as jnp
from jax import lax
from jax.experimental import pallas as pl
from jax.experimental.pallas import tpu as pltpu
from jax.experimental.pallas import tpu_sc as plsc

_CH = 128


def _sc_info():
    try:
        info = plsc.get_sparse_core_info()
        return info.num_cores, info.num_subcores
    except Exception:
        return 2, 16


def _sc_gather(nidx, tidx, uidx, entity_table, user_table, *, nc, ns):
    nw = nc * ns
    n_chunks = nidx.shape[0]
    b_chunks = tidx.shape[0]
    d = entity_table.shape[1]
    assert n_chunks % nw == 0 and b_chunks % nw == 0
    ncw = n_chunks // nw
    bcw = b_chunks // nw

    mesh = plsc.VectorSubcoreMesh(
        core_axis_name="c", subcore_axis_name="s", num_cores=nc,
        num_subcores=ns)

    BK = n_chunks * _CH
    B = b_chunks * _CH

    @functools.partial(
        pl.kernel,
        out_type=[
            jax.ShapeDtypeStruct((BK, d), jnp.float32),
            jax.ShapeDtypeStruct((B, d), jnp.float32),
            jax.ShapeDtypeStruct((B, d), jnp.float32),
        ],
        mesh=mesh,
        compiler_params=pltpu.CompilerParams(use_tc_tiling_on_sc=False),
        scratch_types=[
            pltpu.VMEM((ncw, _CH), jnp.int32),
            pltpu.VMEM((bcw, _CH), jnp.int32),
            pltpu.VMEM((bcw, _CH), jnp.int32),
            pltpu.VMEM((_CH, d), jnp.float32),
            pltpu.SemaphoreType.DMA,
        ],
    )
    def k(nidx_hbm, tidx_hbm, uidx_hbm, etab, utab, out_n, out_t, out_u,
          nidx_v, tidx_v, uidx_v, rows_v, sem):
        wid = lax.axis_index("s") * nc + lax.axis_index("c")

        pltpu.sync_copy(nidx_hbm.at[pl.ds(wid * ncw, ncw)], nidx_v)
        pltpu.sync_copy(tidx_hbm.at[pl.ds(wid * bcw, bcw)], tidx_v)
        pltpu.sync_copy(uidx_hbm.at[pl.ds(wid * bcw, bcw)], uidx_v)

        def run(idx_v, table, out, cpw, _):
            base = wid * cpw * _CH

            def body(j, carry):
                pltpu.async_copy(table.at[idx_v.at[j]], rows_v, sem).wait()
                pltpu.sync_copy(rows_v, out.at[pl.ds(base + j * _CH, _CH)])
                return carry

            lax.fori_loop(0, cpw, body, 0, unroll=False)

        run(nidx_v, etab, out_n, ncw, 0)
        run(tidx_v, etab, out_t, bcw, 1)
        run(uidx_v, utab, out_u, bcw, 2)

    return k(nidx, tidx, uidx, entity_table, user_table)


def _tc_attention_body(tgt_ref, neigh_ref, usr_ref, w_ref, a_ref, out_ref,
                       *, k_neigh, d):
    h = d // 2
    w = w_ref[...]
    a = a_ref[...]
    wa1 = w @ a[:h]
    wa2 = w @ a[h:]

    def maxnorm_scale(x):
        ssq = jnp.sum(x * x, axis=1, keepdims=True)
        return jnp.minimum(lax.rsqrt(ssq), 1.0)

    t = tgt_ref[...]
    th = t * maxnorm_scale(t)
    ts = th @ wa1

    scores = []
    scales = []
    for kk in range(k_neigh):
        nk = neigh_ref[:, kk * d:(kk + 1) * d]
        sck = maxnorm_scale(nk)
        sk = (nk @ wa2) * sck
        e = ts + sk
        scores.append(jnp.where(e > 0, e, 0.2 * e))
        scales.append(sck)
    E = jnp.concatenate(scores, axis=1)
    S = jnp.concatenate(scales, axis=1)
    m = jnp.max(E, axis=1, keepdims=True)
    P = jnp.exp(E - m)
    coef = (P / jnp.sum(P, axis=1, keepdims=True)) * S

    agg = jnp.zeros_like(t)
    for kk in range(k_neigh):
        nk = neigh_ref[:, kk * d:(kk + 1) * d]
        agg = agg + coef[:, kk:kk + 1] * nk
    head = agg @ w

    u = usr_ref[...]
    uh = u * maxnorm_scale(u)
    uhs = uh[:, :h] + uh[:, h:]
    uv = jnp.sum(head * uhs, axis=1, keepdims=True)
    out_ref[...] = 1.0 / (1.0 + jnp.exp(-uv))


def _tc_attention(tgt, neigh, usr, w, a, *, rows):
    b, d = tgt.shape
    k_neigh = neigh.shape[1] // d
    grid = b // rows
    body = functools.partial(_tc_attention_body, k_neigh=k_neigh, d=d)
    out = pl.pallas_call(
        body,
        grid=(grid,),
        in_specs=[
            pl.BlockSpec((rows, d), lambda g: (g, 0)),
            pl.BlockSpec((rows, k_neigh * d), lambda g: (g, 0)),
            pl.BlockSpec((rows, d), lambda g: (g, 0)),
            pl.BlockSpec((d, d // 2), lambda g: (0, 0)),
            pl.BlockSpec((d, 1), lambda g: (0, 0)),
        ],
        out_specs=pl.BlockSpec((rows, 1), lambda g: (g, 0)),
        out_shape=jax.ShapeDtypeStruct((b, 1), jnp.float32),
    )(tgt, neigh, usr, w, a)
    return out.reshape(b)


def kernel(u, i, neighbors, entity_table, user_table, W, a):
    b, k_neigh = neighbors.shape
    d = entity_table.shape[1]
    nc, ns = _sc_info()

    nidx = neighbors.reshape(b * k_neigh // _CH, _CH).astype(jnp.int32)
    tidx = i.reshape(b // _CH, _CH).astype(jnp.int32)
    uidx = u.reshape(b // _CH, _CH).astype(jnp.int32)

    neigh_flat, tgt, usr = _sc_gather(
        nidx, tidx, uidx, entity_table, user_table, nc=nc, ns=ns)
    neigh = neigh_flat.reshape(b, k_neigh * d)
    return _tc_attention(tgt, neigh, usr, W, a, rows=1024)


# speedup vs baseline: 1.7921x; 1.3219x over previous
import functools

import jax
import jax.numpy as jnp
from jax import lax
from jax.experimental import pallas as pl
from jax.experimental.pallas import tpu as pltpu
from jax.experimental.pallas import tpu_sc as plsc

_CH = 128


def _sc_info():
    try:
        info = plsc.get_sparse_core_info()
        return info.num_cores, info.num_subcores
    except Exception:
        return 2, 16


def _sc_gather(nidx, tidx, uidx, entity_table, user_table, *, nc, ns):
    nw = nc * ns
    n_chunks = nidx.shape[0]
    b_chunks = tidx.shape[0]
    d = entity_table.shape[1]
    assert n_chunks % nw == 0 and b_chunks % nw == 0
    ncw = n_chunks // nw
    bcw = b_chunks // nw

    mesh = plsc.VectorSubcoreMesh(
        core_axis_name="c", subcore_axis_name="s", num_cores=nc,
        num_subcores=ns)

    BK = n_chunks * _CH
    B = b_chunks * _CH

    @functools.partial(
        pl.kernel,
        out_type=[
            jax.ShapeDtypeStruct((BK, d), jnp.float32),
            jax.ShapeDtypeStruct((B, d), jnp.float32),
            jax.ShapeDtypeStruct((B, d), jnp.float32),
        ],
        mesh=mesh,
        compiler_params=pltpu.CompilerParams(use_tc_tiling_on_sc=False),
        scratch_types=[
            pltpu.VMEM((ncw, _CH), jnp.int32),
            pltpu.VMEM((bcw, _CH), jnp.int32),
            pltpu.VMEM((bcw, _CH), jnp.int32),
            pltpu.VMEM((_CH, d), jnp.float32),
            pltpu.SemaphoreType.DMA,
        ],
    )
    def k(nidx_hbm, tidx_hbm, uidx_hbm, etab, utab, out_n, out_t, out_u,
          nidx_v, tidx_v, uidx_v, rows_v, sem):
        wid = lax.axis_index("s") * nc + lax.axis_index("c")

        pltpu.sync_copy(nidx_hbm.at[pl.ds(wid * ncw, ncw)], nidx_v)
        pltpu.sync_copy(tidx_hbm.at[pl.ds(wid * bcw, bcw)], tidx_v)
        pltpu.sync_copy(uidx_hbm.at[pl.ds(wid * bcw, bcw)], uidx_v)

        def run(idx_v, table, out, cpw, _):
            base = wid * cpw * _CH

            def body(j, carry):
                pltpu.async_copy(table.at[idx_v.at[j]], rows_v, sem).wait()
                pltpu.sync_copy(rows_v, out.at[pl.ds(base + j * _CH, _CH)])
                return carry

            lax.fori_loop(0, cpw, body, 0, unroll=False)

        run(nidx_v, etab, out_n, ncw, 0)
        run(tidx_v, etab, out_t, bcw, 1)
        run(uidx_v, utab, out_u, bcw, 2)

    return k(nidx, tidx, uidx, entity_table, user_table)


def _tc_attention_body(tgt_ref, neigh_ref, usr_ref, wa1_ref, m_ref, a2m_ref,
                       gw_ref, hs_ref, out_ref):
    t = tgt_ref[...]
    n = neigh_ref[...]
    u = usr_ref[...]

    sst = jnp.sum(t * t, axis=1, keepdims=True)
    sct = jnp.minimum(lax.rsqrt(sst), 1.0)
    ts = (t @ wa1_ref[...]) * sct

    m = m_ref[...]
    ssq = (n * n) @ m
    inv = jnp.minimum(lax.rsqrt(ssq), 1.0)
    e = ts + (n @ a2m_ref[...]) * inv
    e = jnp.where(e > 0, e, 0.2 * e)
    mx = jnp.max(e, axis=1, keepdims=True)
    p = jnp.exp(e - mx)
    coef = (p / jnp.sum(p, axis=1, keepdims=True)) * inv

    cexp = lax.dot_general(coef, m, (((1,), (1,)), ((), ())))
    head = (n * cexp) @ gw_ref[...]

    ssu = jnp.sum(u * u, axis=1, keepdims=True)
    scu = jnp.minimum(lax.rsqrt(ssu), 1.0)
    uhs = (u @ hs_ref[...]) * scu

    uv = jnp.sum(head * uhs, axis=1, keepdims=True)
    out_ref[...] = 1.0 / (1.0 + jnp.exp(-uv))


def _tc_attention(tgt, neigh, usr, w, a, *, rows):
    b, d = tgt.shape
    kd = neigh.shape[1]
    k_neigh = kd // d
    h = d // 2
    f32 = jnp.float32

    wa1 = w @ a[:h]
    wa2 = (w @ a[h:])[:, 0]
    seg = (jnp.arange(kd)[:, None] // d
           == jnp.arange(k_neigh)[None, :]).astype(f32)
    a2m = seg * jnp.tile(wa2, k_neigh)[:, None]
    gw = jnp.tile(w, (k_neigh, 1))
    hs = (jnp.arange(d)[:, None] % h
          == jnp.arange(h)[None, :]).astype(f32)

    grid = b // rows
    out = pl.pallas_call(
        _tc_attention_body,
        grid=(grid,),
        in_specs=[
            pl.BlockSpec((rows, d), lambda g: (g, 0)),
            pl.BlockSpec((rows, kd), lambda g: (g, 0)),
            pl.BlockSpec((rows, d), lambda g: (g, 0)),
            pl.BlockSpec((d, 1), lambda g: (0, 0)),
            pl.BlockSpec((kd, k_neigh), lambda g: (0, 0)),
            pl.BlockSpec((kd, k_neigh), lambda g: (0, 0)),
            pl.BlockSpec((kd, h), lambda g: (0, 0)),
            pl.BlockSpec((d, h), lambda g: (0, 0)),
        ],
        out_specs=pl.BlockSpec((rows, 1), lambda g: (g, 0)),
        out_shape=jax.ShapeDtypeStruct((b, 1), jnp.float32),
    )(tgt, neigh, usr, wa1, seg, a2m, gw, hs)
    return out.reshape(b)


def kernel(u, i, neighbors, entity_table, user_table, W, a):
    b, k_neigh = neighbors.shape
    d = entity_table.shape[1]
    nc, ns = _sc_info()

    nidx = neighbors.reshape(b * k_neigh // _CH, _CH).astype(jnp.int32)
    tidx = i.reshape(b // _CH, _CH).astype(jnp.int32)
    uidx = u.reshape(b // _CH, _CH).astype(jnp.int32)

    neigh_flat, tgt, usr = _sc_gather(
        nidx, tidx, uidx, entity_table, user_table, nc=nc, ns=ns)
    neigh = neigh_flat.reshape(b, k_neigh * d)
    return _tc_attention(tgt, neigh, usr, W, a, rows=1024)
